# baseline (speedup 1.0000x reference)
import functools

import jax
import jax.numpy as jnp
from jax.experimental import pallas as pl
from jax.experimental.pallas import tpu as pltpu

_EPS = 1e-5
_HIGH = jax.lax.Precision.HIGHEST
_F32 = jnp.float32


def _dot(a, b, precision=jax.lax.Precision.DEFAULT):
    return jax.lax.dot_general(a, b, (((1,), (0,)), ((), ())),
                               preferred_element_type=_F32,
                               precision=precision)


def _bf(v):
    return v.astype(jnp.bfloat16).astype(_F32)


def _split3(v):
    hi = _bf(v)
    r = v - hi
    mid = _bf(r)
    return hi, mid, _bf(r - mid)


def _gather_dot(h, parts):
    hi, mid, lo = parts
    return _dot(h, hi) + (_dot(h, mid) + _dot(h, lo))


def _emb_kernel(x_ref, w1_ref, s1_ref, c1_ref, w2_ref, s2_ref, c2_ref, o_ref):
    x = x_ref[...]
    h = _dot(x, w1_ref[...])
    h = jnp.maximum(h * s1_ref[...] + c1_ref[...], 0.0)
    h = _dot(h, w2_ref[...])
    o_ref[...] = jnp.maximum(h * s2_ref[...] + c2_ref[...], 0.0)


def _emb(x_rows, w1t, s1, c1, w2t, s2, c2, tile=1024):
    m = x_rows.shape[0]
    grid = (m // tile,)
    return pl.pallas_call(
        _emb_kernel,
        grid=grid,
        in_specs=[
            pl.BlockSpec((tile, 3), lambda i: (i, 0)),
            pl.BlockSpec(w1t.shape, lambda i: (0, 0)),
            pl.BlockSpec(s1.shape, lambda i: (0, 0)),
            pl.BlockSpec(c1.shape, lambda i: (0, 0)),
            pl.BlockSpec(w2t.shape, lambda i: (0, 0)),
            pl.BlockSpec(s2.shape, lambda i: (0, 0)),
            pl.BlockSpec(c2.shape, lambda i: (0, 0)),
        ],
        out_specs=pl.BlockSpec((tile, 64), lambda i: (i, 0)),
        out_shape=jax.ShapeDtypeStruct((m, 64), _F32),
    )(x_rows, w1t, s1, c1, w2t, s2, c2)


_FPS_WIN = 128


def _fps_kernel(xs_ref, ys_ref, zs_ref, col_ref, wcol_ref,
                idx_ref, cx_ref, cy_ref, cz_ref, *, npoint, n):
    xs = xs_ref[0]
    ys = ys_ref[0]
    zs = zs_ref[0]
    col = col_ref[0]
    wcol = wcol_ref[0]

    wsz = _FPS_WIN
    dist = col * 0.0 + 1e10
    far = col[:1, :1] * 0.0

    for w in range(npoint // wsz):
        zw = wcol * 0.0

        def body(i, carry):
            dist, far, oidx, ocx, ocy, ocz = carry
            m = col == far
            cx = jnp.sum(jnp.where(m, xs, 0.0), axis=(0, 1), keepdims=True)
            cy = jnp.sum(jnp.where(m, ys, 0.0), axis=(0, 1), keepdims=True)
            cz = jnp.sum(jnp.where(m, zs, 0.0), axis=(0, 1), keepdims=True)
            sel = jnp.where(wcol == jnp.float32(0.0) + i.astype(_F32),
                            1.0, 0.0)
            oidx = oidx + sel * far
            ocx = ocx + sel * cx
            ocy = ocy + sel * cy
            ocz = ocz + sel * cz
            dx = xs - cx
            dy = ys - cy
            dz = zs - cz
            d = dx * dx + dy * dy + dz * dz
            dist = jnp.minimum(dist, d)
            mx = jnp.max(dist, axis=(0, 1), keepdims=True)
            far = jnp.min(jnp.where(dist == mx, col, float(n)), axis=(0, 1),
                          keepdims=True)
            return dist, far, oidx, ocx, ocy, ocz

        dist, far, oidx, ocx, ocy, ocz = jax.lax.fori_loop(
            0, wsz, body, (dist, far, zw, zw, zw, zw))
        sl = slice(w * wsz, (w + 1) * wsz)
        idx_ref[0, 0:1, sl] = oidx.astype(jnp.int32)
        cx_ref[0, 0:1, sl] = ocx
        cy_ref[0, 0:1, sl] = ocy
        cz_ref[0, 0:1, sl] = ocz


def _fps(xs, ys, zs, npoint):
    b, n = xs.shape
    n8 = n // 8
    xs3 = xs.reshape(b, 8, n8)
    ys3 = ys.reshape(b, 8, n8)
    zs3 = zs.reshape(b, 8, n8)
    col = jnp.broadcast_to(jnp.arange(n, dtype=_F32).reshape(1, 8, n8),
                           (b, 8, n8))
    wcol = jnp.broadcast_to(jnp.arange(_FPS_WIN, dtype=_F32)[None, None, :],
                            (b, 1, _FPS_WIN))
    out_shapes = (
        jax.ShapeDtypeStruct((b, 1, npoint), jnp.int32),
        jax.ShapeDtypeStruct((b, 1, npoint), _F32),
        jax.ShapeDtypeStruct((b, 1, npoint), _F32),
        jax.ShapeDtypeStruct((b, 1, npoint), _F32),
    )
    specs3 = pl.BlockSpec((1, 8, n8), lambda bb: (bb, 0, 0))
    ospec = pl.BlockSpec((1, 1, npoint), lambda bb: (bb, 0, 0))
    idx, cx, cy, cz = pl.pallas_call(
        functools.partial(_fps_kernel, npoint=npoint, n=n),
        grid=(b,),
        in_specs=[specs3, specs3, specs3, specs3,
                  pl.BlockSpec((1, 1, _FPS_WIN), lambda bb: (bb, 0, 0))],
        out_specs=(ospec, ospec, ospec, ospec),
        out_shape=out_shapes,
        compiler_params=pltpu.CompilerParams(
            dimension_semantics=("parallel",)),
    )(xs3, ys3, zs3, col, wcol)
    return idx[:, 0, :], cx[:, 0, :], cy[:, 0, :], cz[:, 0, :]


def _group_kernel(cxyz_ref, pxyzt_ref, fidx_ref, p_ref,
                  w1_ref, s1_ref, c1_ref, w2_ref, s2_ref, c2_ref, o_ref,
                  *, nsample, radius_sq, sblk, chunk):
    n = pxyzt_ref.shape[2]

    c = cxyz_ref[0]
    cx = c[:, 0:1]
    cy = c[:, 1:2]
    cz = c[:, 2:3]
    px = pxyzt_ref[0, 0:1, :]
    py = pxyzt_ref[0, 1:2, :]
    pz = pxyzt_ref[0, 2:3, :]

    cxb = _bf(cx)
    cyb = _bf(cy)
    czb = _bf(cz)
    pxb = _bf(px)
    pyb = _bf(py)
    pzb = _bf(pz)
    dots = cxb * pxb + cyb * pyb + czb * pzb
    cn = cx * cx + cy * cy + cz * cz
    pn = px * px + py * py + pz * pz
    sqr = (cn + pn) - 2.0 * dots

    mask = sqr <= radius_sq
    mask_f = jnp.where(mask, 1.0, 0.0).astype(_F32)

    tri = (jax.lax.broadcasted_iota(jnp.int32, (chunk, chunk), 0)
           <= jax.lax.broadcasted_iota(jnp.int32, (chunk, chunk), 1))
    tri = tri.astype(_F32)
    offs = jnp.zeros((sblk, 1), _F32)
    ranks = []
    for ci in range(n // chunk):
        mc = mask_f[:, ci * chunk:(ci + 1) * chunk]
        rc = _dot(mc, tri) + offs
        ranks.append(rc)
        offs = rc[:, chunk - 1:chunk]
    rank = jnp.concatenate(ranks, axis=1)
    cnt = offs

    fparts = _split3(p_ref[0])

    fidx = fidx_ref[0]
    col = jax.lax.broadcasted_iota(jnp.int32, (sblk, n), 1)
    hc = jnp.where(col == fidx, 1.0, 0.0).astype(_F32)
    ctr = _gather_dot(hc, fparts)

    rows = []
    for k in range(1, nsample + 1):
        kk = jnp.where(cnt >= float(k), float(k), 1.0)
        hk = jnp.where((rank == kk) & mask, 1.0, 0.0).astype(_F32)
        gk = _gather_dot(hk, fparts)
        rows.append(jnp.concatenate([gk - ctr, ctr], axis=1))
    newp = jnp.concatenate(rows, axis=0)

    h = _dot(newp, w1_ref[...])
    h = jnp.maximum(h * s1_ref[...] + c1_ref[...], 0.0)
    h = _dot(h, w2_ref[...])
    h = jnp.maximum(h * s2_ref[...] + c2_ref[...], 0.0)
    cout = h.shape[1]
    h = h.reshape(nsample, sblk, cout)
    o_ref[0] = jnp.max(h, axis=0)


def _group_stage(new_xyz, xyzt, fidx3, feats, w1t, s1, c1, w2t, s2, c2,
                 nsample, radius_sq, sblk=128, chunk=512):
    b, s, _ = new_xyz.shape
    n = xyzt.shape[2]
    cp = feats.shape[2]
    cout = w2t.shape[1]
    grid = (b, s // sblk)
    return pl.pallas_call(
        functools.partial(_group_kernel, nsample=nsample,
                          radius_sq=radius_sq, sblk=sblk, chunk=chunk),
        grid=grid,
        in_specs=[
            pl.BlockSpec((1, sblk, 3), lambda bb, tt: (bb, tt, 0)),
            pl.BlockSpec((1, 3, n), lambda bb, tt: (bb, 0, 0)),
            pl.BlockSpec((1, sblk, 1), lambda bb, tt: (bb, tt, 0)),
            pl.BlockSpec((1, n, cp), lambda bb, tt: (bb, 0, 0)),
            pl.BlockSpec(w1t.shape, lambda bb, tt: (0, 0)),
            pl.BlockSpec(s1.shape, lambda bb, tt: (0, 0)),
            pl.BlockSpec(c1.shape, lambda bb, tt: (0, 0)),
            pl.BlockSpec(w2t.shape, lambda bb, tt: (0, 0)),
            pl.BlockSpec(s2.shape, lambda bb, tt: (0, 0)),
            pl.BlockSpec(c2.shape, lambda bb, tt: (0, 0)),
        ],
        out_specs=pl.BlockSpec((1, sblk, cout), lambda bb, tt: (bb, tt, 0)),
        out_shape=jax.ShapeDtypeStruct((b, s, cout), _F32),
        compiler_params=pltpu.CompilerParams(
            dimension_semantics=("parallel", "parallel")),
    )(new_xyz, xyzt, fidx3, feats, w1t, s1, c1, w2t, s2, c2)


def _fp_kernel(qxyz_ref, rxyzt_ref, p1_ref, p2_ref, *refs, tq, nref,
               nlayers, has_final_bias):
    layer_refs = refs[:-1]
    o_ref = refs[-1]

    q = qxyz_ref[0]
    qx = q[:, 0:1]
    qy = q[:, 1:2]
    qz = q[:, 2:3]
    rx = rxyzt_ref[0, 0:1, :]
    ry = rxyzt_ref[0, 1:2, :]
    rz = rxyzt_ref[0, 2:3, :]

    qxb = _bf(qx)
    qyb = _bf(qy)
    qzb = _bf(qz)
    rxb = _bf(rx)
    ryb = _bf(ry)
    rzb = _bf(rz)
    dots = qxb * rxb + qyb * ryb + qzb * rzb
    qn = qx * qx + qy * qy + qz * qz
    rn = rx * rx + ry * ry + rz * rz
    sqr = (qn + rn) - 2.0 * dots

    col = jax.lax.broadcasted_iota(jnp.int32, (tq, nref), 1)
    key = sqr
    dvals = []
    idxs = []
    for _ in range(3):
        m = jnp.min(key, axis=1, keepdims=True)
        is_min = key == m
        it = jnp.min(jnp.where(is_min, col, nref), axis=1, keepdims=True)
        dvals.append(m)
        idxs.append(it)
        key = jnp.where(col == it, jnp.inf, key)

    r0 = 1.0 / (dvals[0] + 1e-8)
    r1 = 1.0 / (dvals[1] + 1e-8)
    r2 = 1.0 / (dvals[2] + 1e-8)
    wsum = r0 + r1 + r2
    a = ((r0 / wsum) * jnp.where(col == idxs[0], 1.0, 0.0)
         + (r1 / wsum) * jnp.where(col == idxs[1], 1.0, 0.0)
         + (r2 / wsum) * jnp.where(col == idxs[2], 1.0, 0.0))

    interp = _dot(a.astype(_F32), p2_ref[0], precision=_HIGH)
    h = jnp.concatenate([p1_ref[0], interp], axis=1)

    for li in range(nlayers):
        w_ref, s_ref, c_ref = layer_refs[3 * li:3 * li + 3]
        h = _dot(h, w_ref[...])
        h = h * s_ref[...] + c_ref[...]
        if li < nlayers - 1 or not has_final_bias:
            h = jnp.maximum(h, 0.0)
    o_ref[0] = h


def _fp_stage(qxyz, rxyzt, p1, p2, layers, tq=256, has_final_bias=False):
    b, s1, _ = qxyz.shape
    nref = rxyzt.shape[2]
    c1 = p1.shape[2]
    cout = layers[-1][0].shape[1]
    nlayers = len(layers)
    grid = (b, s1 // tq)
    in_specs = [
        pl.BlockSpec((1, tq, 3), lambda bb, tt: (bb, tt, 0)),
        pl.BlockSpec((1, 3, nref), lambda bb, tt: (bb, 0, 0)),
        pl.BlockSpec((1, tq, c1), lambda bb, tt: (bb, tt, 0)),
        pl.BlockSpec((1, nref, p2.shape[2]), lambda bb, tt: (bb, 0, 0)),
    ]
    args = [qxyz, rxyzt, p1, p2]
    for (wt, sc, cc) in layers:
        for arr in (wt, sc, cc):
            in_specs.append(pl.BlockSpec(arr.shape, lambda bb, tt: (0, 0)))
            args.append(arr)
    return pl.pallas_call(
        functools.partial(_fp_kernel, tq=tq, nref=nref, nlayers=nlayers,
                          has_final_bias=has_final_bias),
        grid=grid,
        in_specs=in_specs,
        out_specs=pl.BlockSpec((1, tq, cout), lambda bb, tt: (bb, tt, 0)),
        out_shape=jax.ShapeDtypeStruct((b, s1, cout), _F32),
        compiler_params=pltpu.CompilerParams(
            dimension_semantics=("parallel", "parallel")),
    )(*args)


def _fold_bn(p, tag):
    s = (p['g_' + tag] / jnp.sqrt(1.0 + _EPS)).reshape(1, -1)
    c = p['b_' + tag].reshape(1, -1)
    return s, c


def kernel(x, params):
    p = params
    b, n, _ = x.shape
    npoint1, npoint2, nsample = 2048, 1024, 32
    radius_sq = 1.0

    xs = x[:, :, 0]
    ys = x[:, :, 1]
    zs = x[:, :, 2]
    xyzt = jnp.transpose(x, (0, 2, 1))

    s1, c1 = _fold_bn(p, 'emb1')
    s2, c2 = _fold_bn(p, 'emb2')
    f0_rows = _emb(x.reshape(b * n, 3), p['w_emb1'].T, s1, c1,
                   p['w_emb2'].T, s2, c2)
    feats0 = f0_rows.reshape(b, n, 64)

    fidx1, c1x, c1y, c1z = _fps(xs, ys, zs, npoint1)
    new_xyz1 = jnp.stack([c1x, c1y, c1z], axis=2)
    s_l0a, c_l0a = _fold_bn(p, 'l0a')
    s_l0b, c_l0b = _fold_bn(p, 'l0b')
    feats1 = _group_stage(new_xyz1, xyzt, fidx1[:, :, None], feats0,
                          p['w_l0a'].T, s_l0a, c_l0a,
                          p['w_l0b'].T, s_l0b, c_l0b,
                          nsample, radius_sq)

    fidx2, c2x, c2y, c2z = _fps(c1x, c1y, c1z, npoint2)
    new_xyz2 = jnp.stack([c2x, c2y, c2z], axis=2)
    xyz1t = jnp.transpose(new_xyz1, (0, 2, 1))
    s_l1a, c_l1a = _fold_bn(p, 'l1a')
    s_l1b, c_l1b = _fold_bn(p, 'l1b')
    feats2 = _group_stage(new_xyz2, xyz1t, fidx2[:, :, None], feats1,
                          p['w_l1a'].T, s_l1a, c_l1a,
                          p['w_l1b'].T, s_l1b, c_l1b,
                          nsample, radius_sq)

    xyz2t = jnp.transpose(new_xyz2, (0, 2, 1))
    s_f0a, c_f0a = _fold_bn(p, 'fp0a')
    s_f0b, c_f0b = _fold_bn(p, 'fp0b')
    fp0 = _fp_stage(new_xyz1, xyz2t, feats1, feats2,
                    [(p['w_fp0a'].T, s_f0a, c_f0a),
                     (p['w_fp0b'].T, s_f0b, c_f0b)])

    s_f1a, c_f1a = _fold_bn(p, 'fp1a')
    s_f1b, c_f1b = _fold_bn(p, 'fp1b')
    s_c1, cc_c1 = _fold_bn(p, 'c1')
    cc_c1 = p['bias_c1'].reshape(1, -1) * s_c1 + cc_c1
    s_c2, cc_c2 = _fold_bn(p, 'c2')
    cc_c2 = p['bias_c2'].reshape(1, -1) * s_c2 + cc_c2
    ones_out = jnp.ones((1, 8), _F32)
    layers = [
        (p['w_fp1a'].T, s_f1a, c_f1a),
        (p['w_fp1b'].T, s_f1b, c_f1b),
        (p['w_c1'].T, s_c1, cc_c1),
        (p['w_c2'].T, s_c2, cc_c2),
        (p['w_out'].T, ones_out, p['bias_out'].reshape(1, -1)),
    ]
    logits = _fp_stage(x, xyz1t, feats0, fp0, layers, has_final_bias=True)
    return logits


# speedup vs baseline: 1.0586x; 1.0586x over previous
import functools

import jax
import jax.numpy as jnp
from jax.experimental import pallas as pl
from jax.experimental.pallas import tpu as pltpu

_EPS = 1e-5
_HIGH = jax.lax.Precision.HIGHEST
_F32 = jnp.float32


def _dot(a, b, precision=jax.lax.Precision.DEFAULT):
    return jax.lax.dot_general(a, b, (((1,), (0,)), ((), ())),
                               preferred_element_type=_F32,
                               precision=precision)


def _bf(v):
    return v.astype(jnp.bfloat16).astype(_F32)


def _split3(v):
    hi = _bf(v)
    r = v - hi
    mid = _bf(r)
    return hi, mid, _bf(r - mid)


def _gather_dot(h, parts):
    hi, mid, lo = parts
    return _dot(h, hi) + (_dot(h, mid) + _dot(h, lo))


def _emb_kernel(x_ref, w1_ref, s1_ref, c1_ref, w2_ref, s2_ref, c2_ref, o_ref):
    x = x_ref[...]
    h = _dot(x, w1_ref[...])
    h = jnp.maximum(h * s1_ref[...] + c1_ref[...], 0.0)
    h = _dot(h, w2_ref[...])
    o_ref[...] = jnp.maximum(h * s2_ref[...] + c2_ref[...], 0.0)


def _emb(x_rows, w1t, s1, c1, w2t, s2, c2, tile=1024):
    m = x_rows.shape[0]
    grid = (m // tile,)
    return pl.pallas_call(
        _emb_kernel,
        grid=grid,
        in_specs=[
            pl.BlockSpec((tile, 3), lambda i: (i, 0)),
            pl.BlockSpec(w1t.shape, lambda i: (0, 0)),
            pl.BlockSpec(s1.shape, lambda i: (0, 0)),
            pl.BlockSpec(c1.shape, lambda i: (0, 0)),
            pl.BlockSpec(w2t.shape, lambda i: (0, 0)),
            pl.BlockSpec(s2.shape, lambda i: (0, 0)),
            pl.BlockSpec(c2.shape, lambda i: (0, 0)),
        ],
        out_specs=pl.BlockSpec((tile, 64), lambda i: (i, 0)),
        out_shape=jax.ShapeDtypeStruct((m, 64), _F32),
    )(x_rows, w1t, s1, c1, w2t, s2, c2)


_FPS_WIN = 128


def _fps_kernel(xs_ref, ys_ref, zs_ref, col_ref, wcol_ref,
                idx_ref, cx_ref, cy_ref, cz_ref, *, npoint, n):
    xs = xs_ref[0]
    ys = ys_ref[0]
    zs = zs_ref[0]
    col = col_ref[0]
    wcol = wcol_ref[0]

    wsz = _FPS_WIN
    dist = col * 0.0 + 1e10
    far = col[:, :1] * 0.0

    for w in range(npoint // wsz):
        zw = wcol * 0.0

        def body(i, carry):
            dist, far, oidx, ocx, ocy, ocz = carry
            m = col == far
            cx = jnp.sum(jnp.where(m, xs, 0.0), axis=1, keepdims=True)
            cy = jnp.sum(jnp.where(m, ys, 0.0), axis=1, keepdims=True)
            cz = jnp.sum(jnp.where(m, zs, 0.0), axis=1, keepdims=True)
            sel = jnp.where(wcol == jnp.float32(0.0) + i.astype(_F32),
                            1.0, 0.0)
            oidx = oidx + sel * far
            ocx = ocx + sel * cx
            ocy = ocy + sel * cy
            ocz = ocz + sel * cz
            dx = xs - cx
            dy = ys - cy
            dz = zs - cz
            d = dx * dx + dy * dy + dz * dz
            dist = jnp.minimum(dist, d)
            mx = jnp.max(dist, axis=1, keepdims=True)
            far = jnp.min(jnp.where(dist == mx, col, float(n)), axis=1,
                          keepdims=True)
            return dist, far, oidx, ocx, ocy, ocz

        dist, far, oidx, ocx, ocy, ocz = jax.lax.fori_loop(
            0, wsz, body, (dist, far, zw, zw, zw, zw))
        sl = slice(w * wsz, (w + 1) * wsz)
        idx_ref[0, 0:1, sl] = oidx.astype(jnp.int32)
        cx_ref[0, 0:1, sl] = ocx
        cy_ref[0, 0:1, sl] = ocy
        cz_ref[0, 0:1, sl] = ocz


def _fps(xs, ys, zs, npoint):
    b, n = xs.shape
    xs3 = xs.reshape(b, 1, n)
    ys3 = ys.reshape(b, 1, n)
    zs3 = zs.reshape(b, 1, n)
    col = jnp.broadcast_to(jnp.arange(n, dtype=_F32)[None, None, :],
                           (b, 1, n))
    wcol = jnp.broadcast_to(jnp.arange(_FPS_WIN, dtype=_F32)[None, None, :],
                            (b, 1, _FPS_WIN))
    out_shapes = (
        jax.ShapeDtypeStruct((b, 1, npoint), jnp.int32),
        jax.ShapeDtypeStruct((b, 1, npoint), _F32),
        jax.ShapeDtypeStruct((b, 1, npoint), _F32),
        jax.ShapeDtypeStruct((b, 1, npoint), _F32),
    )
    specs3 = pl.BlockSpec((1, 1, n), lambda bb: (bb, 0, 0))
    ospec = pl.BlockSpec((1, 1, npoint), lambda bb: (bb, 0, 0))
    idx, cx, cy, cz = pl.pallas_call(
        functools.partial(_fps_kernel, npoint=npoint, n=n),
        grid=(b,),
        in_specs=[specs3, specs3, specs3, specs3,
                  pl.BlockSpec((1, 1, _FPS_WIN), lambda bb: (bb, 0, 0))],
        out_specs=(ospec, ospec, ospec, ospec),
        out_shape=out_shapes,
        compiler_params=pltpu.CompilerParams(
            dimension_semantics=("parallel",)),
    )(xs3, ys3, zs3, col, wcol)
    return idx[:, 0, :], cx[:, 0, :], cy[:, 0, :], cz[:, 0, :]


def _group_kernel(cxyz_ref, pxyzt_ref, fidx_ref, p_ref,
                  w1_ref, s1_ref, c1_ref, w2_ref, s2_ref, c2_ref, o_ref,
                  *, nsample, radius_sq, sblk, chunk):
    n = pxyzt_ref.shape[2]

    c = cxyz_ref[0]
    cx = c[:, 0:1]
    cy = c[:, 1:2]
    cz = c[:, 2:3]
    px = pxyzt_ref[0, 0:1, :]
    py = pxyzt_ref[0, 1:2, :]
    pz = pxyzt_ref[0, 2:3, :]

    cxb = _bf(cx)
    cyb = _bf(cy)
    czb = _bf(cz)
    pxb = _bf(px)
    pyb = _bf(py)
    pzb = _bf(pz)
    dots = cxb * pxb + cyb * pyb + czb * pzb
    cn = cx * cx + cy * cy + cz * cz
    pn = px * px + py * py + pz * pz
    sqr = (cn + pn) - 2.0 * dots

    mask = sqr <= radius_sq
    mask_f = jnp.where(mask, 1.0, 0.0).astype(_F32)

    tri = (jax.lax.broadcasted_iota(jnp.int32, (chunk, chunk), 0)
           <= jax.lax.broadcasted_iota(jnp.int32, (chunk, chunk), 1))
    tri = tri.astype(_F32)
    offs = jnp.zeros((sblk, 1), _F32)
    ranks = []
    for ci in range(n // chunk):
        mc = mask_f[:, ci * chunk:(ci + 1) * chunk]
        rc = _dot(mc, tri) + offs
        ranks.append(rc)
        offs = rc[:, chunk - 1:chunk]
    rank = jnp.concatenate(ranks, axis=1)
    cnt = offs

    fparts = _split3(p_ref[0])

    fidx = fidx_ref[0]
    col = jax.lax.broadcasted_iota(jnp.int32, (sblk, n), 1)
    hc = jnp.where(col == fidx, 1.0, 0.0).astype(_F32)
    ctr = _gather_dot(hc, fparts)

    rows = []
    for k in range(1, nsample + 1):
        kk = jnp.where(cnt >= float(k), float(k), 1.0)
        hk = jnp.where((rank == kk) & mask, 1.0, 0.0).astype(_F32)
        gk = _gather_dot(hk, fparts)
        rows.append(jnp.concatenate([gk - ctr, ctr], axis=1))
    newp = jnp.concatenate(rows, axis=0)

    h = _dot(newp, w1_ref[...])
    h = jnp.maximum(h * s1_ref[...] + c1_ref[...], 0.0)
    h = _dot(h, w2_ref[...])
    h = jnp.maximum(h * s2_ref[...] + c2_ref[...], 0.0)
    cout = h.shape[1]
    h = h.reshape(nsample, sblk, cout)
    o_ref[0] = jnp.max(h, axis=0)


def _group_stage(new_xyz, xyzt, fidx3, feats, w1t, s1, c1, w2t, s2, c2,
                 nsample, radius_sq, sblk=128, chunk=512):
    b, s, _ = new_xyz.shape
    n = xyzt.shape[2]
    cp = feats.shape[2]
    cout = w2t.shape[1]
    grid = (b, s // sblk)
    return pl.pallas_call(
        functools.partial(_group_kernel, nsample=nsample,
                          radius_sq=radius_sq, sblk=sblk, chunk=chunk),
        grid=grid,
        in_specs=[
            pl.BlockSpec((1, sblk, 3), lambda bb, tt: (bb, tt, 0)),
            pl.BlockSpec((1, 3, n), lambda bb, tt: (bb, 0, 0)),
            pl.BlockSpec((1, sblk, 1), lambda bb, tt: (bb, tt, 0)),
            pl.BlockSpec((1, n, cp), lambda bb, tt: (bb, 0, 0)),
            pl.BlockSpec(w1t.shape, lambda bb, tt: (0, 0)),
            pl.BlockSpec(s1.shape, lambda bb, tt: (0, 0)),
            pl.BlockSpec(c1.shape, lambda bb, tt: (0, 0)),
            pl.BlockSpec(w2t.shape, lambda bb, tt: (0, 0)),
            pl.BlockSpec(s2.shape, lambda bb, tt: (0, 0)),
            pl.BlockSpec(c2.shape, lambda bb, tt: (0, 0)),
        ],
        out_specs=pl.BlockSpec((1, sblk, cout), lambda bb, tt: (bb, tt, 0)),
        out_shape=jax.ShapeDtypeStruct((b, s, cout), _F32),
        compiler_params=pltpu.CompilerParams(
            dimension_semantics=("parallel", "parallel")),
    )(new_xyz, xyzt, fidx3, feats, w1t, s1, c1, w2t, s2, c2)


def _fp_kernel(qxyz_ref, rxyzt_ref, p1_ref, p2_ref, *refs, tq, nref,
               nlayers, has_final_bias):
    layer_refs = refs[:-1]
    o_ref = refs[-1]

    q = qxyz_ref[0]
    qx = q[:, 0:1]
    qy = q[:, 1:2]
    qz = q[:, 2:3]
    rx = rxyzt_ref[0, 0:1, :]
    ry = rxyzt_ref[0, 1:2, :]
    rz = rxyzt_ref[0, 2:3, :]

    qxb = _bf(qx)
    qyb = _bf(qy)
    qzb = _bf(qz)
    rxb = _bf(rx)
    ryb = _bf(ry)
    rzb = _bf(rz)
    dots = qxb * rxb + qyb * ryb + qzb * rzb
    qn = qx * qx + qy * qy + qz * qz
    rn = rx * rx + ry * ry + rz * rz
    sqr = (qn + rn) - 2.0 * dots

    col = jax.lax.broadcasted_iota(jnp.int32, (tq, nref), 1)
    key = sqr
    dvals = []
    idxs = []
    for _ in range(3):
        m = jnp.min(key, axis=1, keepdims=True)
        is_min = key == m
        it = jnp.min(jnp.where(is_min, col, nref), axis=1, keepdims=True)
        dvals.append(m)
        idxs.append(it)
        key = jnp.where(col == it, jnp.inf, key)

    r0 = 1.0 / (dvals[0] + 1e-8)
    r1 = 1.0 / (dvals[1] + 1e-8)
    r2 = 1.0 / (dvals[2] + 1e-8)
    wsum = r0 + r1 + r2
    a = ((r0 / wsum) * jnp.where(col == idxs[0], 1.0, 0.0)
         + (r1 / wsum) * jnp.where(col == idxs[1], 1.0, 0.0)
         + (r2 / wsum) * jnp.where(col == idxs[2], 1.0, 0.0))

    interp = _dot(a.astype(_F32), p2_ref[0], precision=_HIGH)
    h = jnp.concatenate([p1_ref[0], interp], axis=1)

    for li in range(nlayers):
        w_ref, s_ref, c_ref = layer_refs[3 * li:3 * li + 3]
        h = _dot(h, w_ref[...])
        h = h * s_ref[...] + c_ref[...]
        if li < nlayers - 1 or not has_final_bias:
            h = jnp.maximum(h, 0.0)
    o_ref[0] = h


def _fp_stage(qxyz, rxyzt, p1, p2, layers, tq=256, has_final_bias=False):
    b, s1, _ = qxyz.shape
    nref = rxyzt.shape[2]
    c1 = p1.shape[2]
    cout = layers[-1][0].shape[1]
    nlayers = len(layers)
    grid = (b, s1 // tq)
    in_specs = [
        pl.BlockSpec((1, tq, 3), lambda bb, tt: (bb, tt, 0)),
        pl.BlockSpec((1, 3, nref), lambda bb, tt: (bb, 0, 0)),
        pl.BlockSpec((1, tq, c1), lambda bb, tt: (bb, tt, 0)),
        pl.BlockSpec((1, nref, p2.shape[2]), lambda bb, tt: (bb, 0, 0)),
    ]
    args = [qxyz, rxyzt, p1, p2]
    for (wt, sc, cc) in layers:
        for arr in (wt, sc, cc):
            in_specs.append(pl.BlockSpec(arr.shape, lambda bb, tt: (0, 0)))
            args.append(arr)
    return pl.pallas_call(
        functools.partial(_fp_kernel, tq=tq, nref=nref, nlayers=nlayers,
                          has_final_bias=has_final_bias),
        grid=grid,
        in_specs=in_specs,
        out_specs=pl.BlockSpec((1, tq, cout), lambda bb, tt: (bb, tt, 0)),
        out_shape=jax.ShapeDtypeStruct((b, s1, cout), _F32),
        compiler_params=pltpu.CompilerParams(
            dimension_semantics=("parallel", "parallel")),
    )(*args)


def _fold_bn(p, tag):
    s = (p['g_' + tag] / jnp.sqrt(1.0 + _EPS)).reshape(1, -1)
    c = p['b_' + tag].reshape(1, -1)
    return s, c


def kernel(x, params):
    p = params
    b, n, _ = x.shape
    npoint1, npoint2, nsample = 2048, 1024, 32
    radius_sq = 1.0

    xs = x[:, :, 0]
    ys = x[:, :, 1]
    zs = x[:, :, 2]
    xyzt = jnp.transpose(x, (0, 2, 1))

    s1, c1 = _fold_bn(p, 'emb1')
    s2, c2 = _fold_bn(p, 'emb2')
    f0_rows = _emb(x.reshape(b * n, 3), p['w_emb1'].T, s1, c1,
                   p['w_emb2'].T, s2, c2)
    feats0 = f0_rows.reshape(b, n, 64)

    fidx1, c1x, c1y, c1z = _fps(xs, ys, zs, npoint1)
    new_xyz1 = jnp.stack([c1x, c1y, c1z], axis=2)
    s_l0a, c_l0a = _fold_bn(p, 'l0a')
    s_l0b, c_l0b = _fold_bn(p, 'l0b')
    feats1 = _group_stage(new_xyz1, xyzt, fidx1[:, :, None], feats0,
                          p['w_l0a'].T, s_l0a, c_l0a,
                          p['w_l0b'].T, s_l0b, c_l0b,
                          nsample, radius_sq)

    fidx2, c2x, c2y, c2z = _fps(c1x, c1y, c1z, npoint2)
    new_xyz2 = jnp.stack([c2x, c2y, c2z], axis=2)
    xyz1t = jnp.transpose(new_xyz1, (0, 2, 1))
    s_l1a, c_l1a = _fold_bn(p, 'l1a')
    s_l1b, c_l1b = _fold_bn(p, 'l1b')
    feats2 = _group_stage(new_xyz2, xyz1t, fidx2[:, :, None], feats1,
                          p['w_l1a'].T, s_l1a, c_l1a,
                          p['w_l1b'].T, s_l1b, c_l1b,
                          nsample, radius_sq)

    xyz2t = jnp.transpose(new_xyz2, (0, 2, 1))
    s_f0a, c_f0a = _fold_bn(p, 'fp0a')
    s_f0b, c_f0b = _fold_bn(p, 'fp0b')
    fp0 = _fp_stage(new_xyz1, xyz2t, feats1, feats2,
                    [(p['w_fp0a'].T, s_f0a, c_f0a),
                     (p['w_fp0b'].T, s_f0b, c_f0b)])

    s_f1a, c_f1a = _fold_bn(p, 'fp1a')
    s_f1b, c_f1b = _fold_bn(p, 'fp1b')
    s_c1, cc_c1 = _fold_bn(p, 'c1')
    cc_c1 = p['bias_c1'].reshape(1, -1) * s_c1 + cc_c1
    s_c2, cc_c2 = _fold_bn(p, 'c2')
    cc_c2 = p['bias_c2'].reshape(1, -1) * s_c2 + cc_c2
    ones_out = jnp.ones((1, 8), _F32)
    layers = [
        (p['w_fp1a'].T, s_f1a, c_f1a),
        (p['w_fp1b'].T, s_f1b, c_f1b),
        (p['w_c1'].T, s_c1, cc_c1),
        (p['w_c2'].T, s_c2, cc_c2),
        (p['w_out'].T, ones_out, p['bias_out'].reshape(1, -1)),
    ]
    logits = _fp_stage(x, xyz1t, feats0, fp0, layers, has_final_bias=True)
    return logits


# speedup vs baseline: 1.4751x; 1.3934x over previous
import functools

import jax
import jax.numpy as jnp
from jax.experimental import pallas as pl
from jax.experimental.pallas import tpu as pltpu

_EPS = 1e-5
_HIGH = jax.lax.Precision.HIGHEST
_F32 = jnp.float32


def _dot(a, b, precision=jax.lax.Precision.DEFAULT):
    return jax.lax.dot_general(a, b, (((1,), (0,)), ((), ())),
                               preferred_element_type=_F32,
                               precision=precision)


def _bf(v):
    return v.astype(jnp.bfloat16).astype(_F32)


def _split3(v):
    hi = _bf(v)
    r = v - hi
    mid = _bf(r)
    return hi, mid, _bf(r - mid)


def _gather_dot(h, parts):
    hi, mid, lo = parts
    return _dot(h, hi) + (_dot(h, mid) + _dot(h, lo))


def _emb_kernel(x_ref, w1_ref, s1_ref, c1_ref, w2_ref, s2_ref, c2_ref, o_ref):
    x = x_ref[...]
    h = _dot(x, w1_ref[...])
    h = jnp.maximum(h * s1_ref[...] + c1_ref[...], 0.0)
    h = _dot(h, w2_ref[...])
    o_ref[...] = jnp.maximum(h * s2_ref[...] + c2_ref[...], 0.0)


def _emb(x_rows, w1t, s1, c1, w2t, s2, c2, tile=1024):
    m = x_rows.shape[0]
    grid = (m // tile,)
    return pl.pallas_call(
        _emb_kernel,
        grid=grid,
        in_specs=[
            pl.BlockSpec((tile, 3), lambda i: (i, 0)),
            pl.BlockSpec(w1t.shape, lambda i: (0, 0)),
            pl.BlockSpec(s1.shape, lambda i: (0, 0)),
            pl.BlockSpec(c1.shape, lambda i: (0, 0)),
            pl.BlockSpec(w2t.shape, lambda i: (0, 0)),
            pl.BlockSpec(s2.shape, lambda i: (0, 0)),
            pl.BlockSpec(c2.shape, lambda i: (0, 0)),
        ],
        out_specs=pl.BlockSpec((tile, 64), lambda i: (i, 0)),
        out_shape=jax.ShapeDtypeStruct((m, 64), _F32),
    )(x_rows, w1t, s1, c1, w2t, s2, c2)


_FPS_WIN = 128


def _fps_kernel(xs_ref, ys_ref, zs_ref, col_ref, wcol_ref,
                idx_ref, cx_ref, cy_ref, cz_ref, *, npoint, n):
    xs = xs_ref[...]
    ys = ys_ref[...]
    zs = zs_ref[...]
    col = col_ref[...]
    wcol = wcol_ref[...]

    wsz = _FPS_WIN
    dist = col * 0.0 + 1e10
    far = col[:, :1] * 0.0

    for w in range(npoint // wsz):
        zw = wcol * 0.0

        def body(i, carry):
            dist, far, oidx, ocx, ocy, ocz = carry
            m = col == far
            cx = jnp.sum(jnp.where(m, xs, 0.0), axis=1, keepdims=True)
            cy = jnp.sum(jnp.where(m, ys, 0.0), axis=1, keepdims=True)
            cz = jnp.sum(jnp.where(m, zs, 0.0), axis=1, keepdims=True)
            sel = jnp.where(wcol == jnp.float32(0.0) + i.astype(_F32),
                            1.0, 0.0)
            oidx = oidx + sel * far
            ocx = ocx + sel * cx
            ocy = ocy + sel * cy
            ocz = ocz + sel * cz
            dx = xs - cx
            dy = ys - cy
            dz = zs - cz
            d = dx * dx + dy * dy + dz * dz
            dist = jnp.minimum(dist, d)
            mx = jnp.max(dist, axis=1, keepdims=True)
            far = jnp.min(jnp.where(dist == mx, col, float(n)), axis=1,
                          keepdims=True)
            return dist, far, oidx, ocx, ocy, ocz

        dist, far, oidx, ocx, ocy, ocz = jax.lax.fori_loop(
            0, wsz, body, (dist, far, zw, zw, zw, zw))
        sl = slice(w * wsz, (w + 1) * wsz)
        idx_ref[:, sl] = oidx.astype(jnp.int32)
        cx_ref[:, sl] = ocx
        cy_ref[:, sl] = ocy
        cz_ref[:, sl] = ocz


def _fps(xs, ys, zs, npoint):
    b, n = xs.shape
    col = jnp.broadcast_to(jnp.arange(n, dtype=_F32)[None, :], (b, n))
    wcol = jnp.broadcast_to(jnp.arange(_FPS_WIN, dtype=_F32)[None, :],
                            (b, _FPS_WIN))
    out_shapes = (
        jax.ShapeDtypeStruct((b, npoint), jnp.int32),
        jax.ShapeDtypeStruct((b, npoint), _F32),
        jax.ShapeDtypeStruct((b, npoint), _F32),
        jax.ShapeDtypeStruct((b, npoint), _F32),
    )
    return pl.pallas_call(
        functools.partial(_fps_kernel, npoint=npoint, n=n),
        out_shape=out_shapes,
    )(xs, ys, zs, col, wcol)


def _group_kernel(cxyz_ref, pxyzt_ref, fidx_ref, p_ref,
                  w1_ref, s1_ref, c1_ref, w2_ref, s2_ref, c2_ref, o_ref,
                  *, nsample, radius_sq, sblk, chunk):
    n = pxyzt_ref.shape[2]

    c = cxyz_ref[0]
    cx = c[:, 0:1]
    cy = c[:, 1:2]
    cz = c[:, 2:3]
    px = pxyzt_ref[0, 0:1, :]
    py = pxyzt_ref[0, 1:2, :]
    pz = pxyzt_ref[0, 2:3, :]

    cxb = _bf(cx)
    cyb = _bf(cy)
    czb = _bf(cz)
    pxb = _bf(px)
    pyb = _bf(py)
    pzb = _bf(pz)
    dots = cxb * pxb + cyb * pyb + czb * pzb
    cn = cx * cx + cy * cy + cz * cz
    pn = px * px + py * py + pz * pz
    sqr = (cn + pn) - 2.0 * dots

    mask = sqr <= radius_sq
    mask_f = jnp.where(mask, 1.0, 0.0).astype(_F32)

    tri = (jax.lax.broadcasted_iota(jnp.int32, (chunk, chunk), 0)
           <= jax.lax.broadcasted_iota(jnp.int32, (chunk, chunk), 1))
    tri = tri.astype(_F32)
    offs = jnp.zeros((sblk, 1), _F32)
    ranks = []
    for ci in range(n // chunk):
        mc = mask_f[:, ci * chunk:(ci + 1) * chunk]
        rc = _dot(mc, tri) + offs
        ranks.append(rc)
        offs = rc[:, chunk - 1:chunk]
    rank = jnp.concatenate(ranks, axis=1)
    cnt = offs

    fparts = _split3(p_ref[0])

    fidx = fidx_ref[0]
    col = jax.lax.broadcasted_iota(jnp.int32, (sblk, n), 1)
    hc = jnp.where(col == fidx, 1.0, 0.0).astype(_F32)
    ctr = _gather_dot(hc, fparts)

    rows = []
    for k in range(1, nsample + 1):
        kk = jnp.where(cnt >= float(k), float(k), 1.0)
        hk = jnp.where((rank == kk) & mask, 1.0, 0.0).astype(_F32)
        gk = _gather_dot(hk, fparts)
        rows.append(jnp.concatenate([gk - ctr, ctr], axis=1))
    newp = jnp.concatenate(rows, axis=0)

    h = _dot(newp, w1_ref[...])
    h = jnp.maximum(h * s1_ref[...] + c1_ref[...], 0.0)
    h = _dot(h, w2_ref[...])
    h = jnp.maximum(h * s2_ref[...] + c2_ref[...], 0.0)
    cout = h.shape[1]
    h = h.reshape(nsample, sblk, cout)
    o_ref[0] = jnp.max(h, axis=0)


def _group_stage(new_xyz, xyzt, fidx3, feats, w1t, s1, c1, w2t, s2, c2,
                 nsample, radius_sq, sblk=128, chunk=512):
    b, s, _ = new_xyz.shape
    n = xyzt.shape[2]
    cp = feats.shape[2]
    cout = w2t.shape[1]
    grid = (b, s // sblk)
    return pl.pallas_call(
        functools.partial(_group_kernel, nsample=nsample,
                          radius_sq=radius_sq, sblk=sblk, chunk=chunk),
        grid=grid,
        in_specs=[
            pl.BlockSpec((1, sblk, 3), lambda bb, tt: (bb, tt, 0)),
            pl.BlockSpec((1, 3, n), lambda bb, tt: (bb, 0, 0)),
            pl.BlockSpec((1, sblk, 1), lambda bb, tt: (bb, tt, 0)),
            pl.BlockSpec((1, n, cp), lambda bb, tt: (bb, 0, 0)),
            pl.BlockSpec(w1t.shape, lambda bb, tt: (0, 0)),
            pl.BlockSpec(s1.shape, lambda bb, tt: (0, 0)),
            pl.BlockSpec(c1.shape, lambda bb, tt: (0, 0)),
            pl.BlockSpec(w2t.shape, lambda bb, tt: (0, 0)),
            pl.BlockSpec(s2.shape, lambda bb, tt: (0, 0)),
            pl.BlockSpec(c2.shape, lambda bb, tt: (0, 0)),
        ],
        out_specs=pl.BlockSpec((1, sblk, cout), lambda bb, tt: (bb, tt, 0)),
        out_shape=jax.ShapeDtypeStruct((b, s, cout), _F32),
        compiler_params=pltpu.CompilerParams(
            dimension_semantics=("parallel", "parallel")),
    )(new_xyz, xyzt, fidx3, feats, w1t, s1, c1, w2t, s2, c2)


def _fp_kernel(qxyz_ref, rxyzt_ref, p1_ref, p2_ref, *refs, tq, nref,
               nlayers, has_final_bias):
    layer_refs = refs[:-1]
    o_ref = refs[-1]

    q = qxyz_ref[0]
    qx = q[:, 0:1]
    qy = q[:, 1:2]
    qz = q[:, 2:3]
    rx = rxyzt_ref[0, 0:1, :]
    ry = rxyzt_ref[0, 1:2, :]
    rz = rxyzt_ref[0, 2:3, :]

    qxb = _bf(qx)
    qyb = _bf(qy)
    qzb = _bf(qz)
    rxb = _bf(rx)
    ryb = _bf(ry)
    rzb = _bf(rz)
    dots = qxb * rxb + qyb * ryb + qzb * rzb
    qn = qx * qx + qy * qy + qz * qz
    rn = rx * rx + ry * ry + rz * rz
    sqr = (qn + rn) - 2.0 * dots

    col = jax.lax.broadcasted_iota(jnp.int32, (tq, nref), 1)
    key = sqr
    dvals = []
    idxs = []
    for _ in range(3):
        m = jnp.min(key, axis=1, keepdims=True)
        is_min = key == m
        it = jnp.min(jnp.where(is_min, col, nref), axis=1, keepdims=True)
        dvals.append(m)
        idxs.append(it)
        key = jnp.where(col == it, jnp.inf, key)

    r0 = 1.0 / (dvals[0] + 1e-8)
    r1 = 1.0 / (dvals[1] + 1e-8)
    r2 = 1.0 / (dvals[2] + 1e-8)
    wsum = r0 + r1 + r2
    a = ((r0 / wsum) * jnp.where(col == idxs[0], 1.0, 0.0)
         + (r1 / wsum) * jnp.where(col == idxs[1], 1.0, 0.0)
         + (r2 / wsum) * jnp.where(col == idxs[2], 1.0, 0.0))

    interp = _dot(a.astype(_F32), p2_ref[0], precision=_HIGH)
    h = jnp.concatenate([p1_ref[0], interp], axis=1)

    for li in range(nlayers):
        w_ref, s_ref, c_ref = layer_refs[3 * li:3 * li + 3]
        h = _dot(h, w_ref[...])
        h = h * s_ref[...] + c_ref[...]
        if li < nlayers - 1 or not has_final_bias:
            h = jnp.maximum(h, 0.0)
    o_ref[0] = h


def _fp_stage(qxyz, rxyzt, p1, p2, layers, tq=256, has_final_bias=False):
    b, s1, _ = qxyz.shape
    nref = rxyzt.shape[2]
    c1 = p1.shape[2]
    cout = layers[-1][0].shape[1]
    nlayers = len(layers)
    grid = (b, s1 // tq)
    in_specs = [
        pl.BlockSpec((1, tq, 3), lambda bb, tt: (bb, tt, 0)),
        pl.BlockSpec((1, 3, nref), lambda bb, tt: (bb, 0, 0)),
        pl.BlockSpec((1, tq, c1), lambda bb, tt: (bb, tt, 0)),
        pl.BlockSpec((1, nref, p2.shape[2]), lambda bb, tt: (bb, 0, 0)),
    ]
    args = [qxyz, rxyzt, p1, p2]
    for (wt, sc, cc) in layers:
        for arr in (wt, sc, cc):
            in_specs.append(pl.BlockSpec(arr.shape, lambda bb, tt: (0, 0)))
            args.append(arr)
    return pl.pallas_call(
        functools.partial(_fp_kernel, tq=tq, nref=nref, nlayers=nlayers,
                          has_final_bias=has_final_bias),
        grid=grid,
        in_specs=in_specs,
        out_specs=pl.BlockSpec((1, tq, cout), lambda bb, tt: (bb, tt, 0)),
        out_shape=jax.ShapeDtypeStruct((b, s1, cout), _F32),
        compiler_params=pltpu.CompilerParams(
            dimension_semantics=("parallel", "parallel")),
    )(*args)


def _fold_bn(p, tag):
    s = (p['g_' + tag] / jnp.sqrt(1.0 + _EPS)).reshape(1, -1)
    c = p['b_' + tag].reshape(1, -1)
    return s, c


def kernel(x, params):
    p = params
    b, n, _ = x.shape
    npoint1, npoint2, nsample = 2048, 1024, 32
    radius_sq = 1.0

    xs = x[:, :, 0]
    ys = x[:, :, 1]
    zs = x[:, :, 2]
    xyzt = jnp.transpose(x, (0, 2, 1))

    s1, c1 = _fold_bn(p, 'emb1')
    s2, c2 = _fold_bn(p, 'emb2')
    f0_rows = _emb(x.reshape(b * n, 3), p['w_emb1'].T, s1, c1,
                   p['w_emb2'].T, s2, c2)
    feats0 = f0_rows.reshape(b, n, 64)

    fidx1, c1x, c1y, c1z = _fps(xs, ys, zs, npoint1)
    new_xyz1 = jnp.stack([c1x, c1y, c1z], axis=2)
    s_l0a, c_l0a = _fold_bn(p, 'l0a')
    s_l0b, c_l0b = _fold_bn(p, 'l0b')
    feats1 = _group_stage(new_xyz1, xyzt, fidx1[:, :, None], feats0,
                          p['w_l0a'].T, s_l0a, c_l0a,
                          p['w_l0b'].T, s_l0b, c_l0b,
                          nsample, radius_sq)

    fidx2, c2x, c2y, c2z = _fps(c1x, c1y, c1z, npoint2)
    new_xyz2 = jnp.stack([c2x, c2y, c2z], axis=2)
    xyz1t = jnp.transpose(new_xyz1, (0, 2, 1))
    s_l1a, c_l1a = _fold_bn(p, 'l1a')
    s_l1b, c_l1b = _fold_bn(p, 'l1b')
    feats2 = _group_stage(new_xyz2, xyz1t, fidx2[:, :, None], feats1,
                          p['w_l1a'].T, s_l1a, c_l1a,
                          p['w_l1b'].T, s_l1b, c_l1b,
                          nsample, radius_sq)

    xyz2t = jnp.transpose(new_xyz2, (0, 2, 1))
    s_f0a, c_f0a = _fold_bn(p, 'fp0a')
    s_f0b, c_f0b = _fold_bn(p, 'fp0b')
    fp0 = _fp_stage(new_xyz1, xyz2t, feats1, feats2,
                    [(p['w_fp0a'].T, s_f0a, c_f0a),
                     (p['w_fp0b'].T, s_f0b, c_f0b)])

    s_f1a, c_f1a = _fold_bn(p, 'fp1a')
    s_f1b, c_f1b = _fold_bn(p, 'fp1b')
    s_c1, cc_c1 = _fold_bn(p, 'c1')
    cc_c1 = p['bias_c1'].reshape(1, -1) * s_c1 + cc_c1
    s_c2, cc_c2 = _fold_bn(p, 'c2')
    cc_c2 = p['bias_c2'].reshape(1, -1) * s_c2 + cc_c2
    ones_out = jnp.ones((1, 8), _F32)
    layers = [
        (p['w_fp1a'].T, s_f1a, c_f1a),
        (p['w_fp1b'].T, s_f1b, c_f1b),
        (p['w_c1'].T, s_c1, cc_c1),
        (p['w_c2'].T, s_c2, cc_c2),
        (p['w_out'].T, ones_out, p['bias_out'].reshape(1, -1)),
    ]
    logits = _fp_stage(x, xyz1t, feats0, fp0, layers, has_final_bias=True)
    return logits
